# baseline (speedup 1.0000x reference)
import functools

import jax
import jax.numpy as jnp
from jax.experimental import pallas as pl


def _fused_cell(x_ref, h_ref, wt_ref, ut_ref, bias_ref, g_ref, b_ref, o_ref):
    pre = jnp.dot(x_ref[...], wt_ref[...], preferred_element_type=jnp.float32)
    pre += jnp.dot(h_ref[...], ut_ref[...], preferred_element_type=jnp.float32)
    pre += bias_ref[...]
    mu = jnp.mean(pre, axis=-1, keepdims=True)
    ctr = pre - mu
    var = jnp.mean(ctr * ctr, axis=-1, keepdims=True)
    normed = ctr * jax.lax.rsqrt(var + 1e-5) * g_ref[...] + b_ref[...]
    o_ref[...] = jax.nn.sigmoid(normed)


@functools.partial(jax.jit, static_argnames=("block_b",))
def _run(x, h_prev, wt, ut, bias, g, b, block_b):
    B, D_in = x.shape
    H = ut.shape[0]
    grid = (B // block_b,)
    return pl.pallas_call(
        _fused_cell,
        grid=grid,
        in_specs=[
            pl.BlockSpec((block_b, D_in), lambda i: (i, 0)),
            pl.BlockSpec((block_b, H), lambda i: (i, 0)),
            pl.BlockSpec((D_in, H), lambda i: (0, 0)),
            pl.BlockSpec((H, H), lambda i: (0, 0)),
            pl.BlockSpec((1, H), lambda i: (0, 0)),
            pl.BlockSpec((1, H), lambda i: (0, 0)),
            pl.BlockSpec((1, H), lambda i: (0, 0)),
        ],
        out_specs=pl.BlockSpec((block_b, H), lambda i: (i, 0)),
        out_shape=jax.ShapeDtypeStruct((B, H), jnp.float32),
    )(x, h_prev, wt, ut, bias, g, b)


def kernel(x, h_prev, W_w, W_b, U_w, U_b, Q_w, Q_b, M_w, M_b, ln_g, ln_b):
    bias = (W_b + U_b + Q_b).reshape(1, -1)
    return _run(
        x,
        h_prev,
        W_w.T,
        U_w.T,
        bias,
        ln_g.reshape(1, -1),
        ln_b.reshape(1, -1),
        block_b=1024,
    )


# speedup vs baseline: 1.2965x; 1.2965x over previous
import functools

import jax
import jax.numpy as jnp
from jax.experimental import pallas as pl


def _fused_cell(x_ref, h_ref, wt_ref, ut_ref, bias_ref, g_ref, b_ref, o_ref):
    pre = jnp.dot(x_ref[...], wt_ref[...], preferred_element_type=jnp.float32)
    pre += jnp.dot(h_ref[...], ut_ref[...], preferred_element_type=jnp.float32)
    pre += bias_ref[...]
    mu = jnp.mean(pre, axis=-1, keepdims=True)
    ctr = pre - mu
    var = jnp.mean(ctr * ctr, axis=-1, keepdims=True)
    normed = ctr * jax.lax.rsqrt(var + 1e-5) * g_ref[...] + b_ref[...]
    o_ref[...] = jax.nn.sigmoid(normed)


@functools.partial(jax.jit, static_argnames=("block_b",))
def _run(x, h_prev, wt, ut, bias, g, b, block_b):
    B, D_in = x.shape
    H = ut.shape[0]
    grid = (B // block_b,)
    return pl.pallas_call(
        _fused_cell,
        grid=grid,
        in_specs=[
            pl.BlockSpec((block_b, D_in), lambda i: (i, 0)),
            pl.BlockSpec((block_b, H), lambda i: (i, 0)),
            pl.BlockSpec((D_in, H), lambda i: (0, 0)),
            pl.BlockSpec((H, H), lambda i: (0, 0)),
            pl.BlockSpec((1, H), lambda i: (0, 0)),
            pl.BlockSpec((1, H), lambda i: (0, 0)),
            pl.BlockSpec((1, H), lambda i: (0, 0)),
        ],
        out_specs=pl.BlockSpec((block_b, H), lambda i: (i, 0)),
        out_shape=jax.ShapeDtypeStruct((B, H), jnp.float32),
    )(x, h_prev, wt, ut, bias, g, b)


def kernel(x, h_prev, W_w, W_b, U_w, U_b, Q_w, Q_b, M_w, M_b, ln_g, ln_b):
    bias = (W_b + U_b + Q_b).reshape(1, -1)
    return _run(
        x,
        h_prev,
        W_w.T,
        U_w.T,
        bias,
        ln_g.reshape(1, -1),
        ln_b.reshape(1, -1),
        block_b=4096,
    )


# speedup vs baseline: 1.3045x; 1.0061x over previous
import functools

import jax
import jax.numpy as jnp
from jax.experimental import pallas as pl


def _fused_cell(x_ref, h_ref, wt_ref, ut_ref, bias_ref, g_ref, b_ref, o_ref):
    pre = jnp.dot(x_ref[...], wt_ref[...], preferred_element_type=jnp.float32)
    pre += jnp.dot(h_ref[...], ut_ref[...], preferred_element_type=jnp.float32)
    pre += bias_ref[...]
    mu = jnp.mean(pre, axis=-1, keepdims=True)
    ctr = pre - mu
    var = jnp.mean(ctr * ctr, axis=-1, keepdims=True)
    normed = ctr * jax.lax.rsqrt(var + 1e-5) * g_ref[...] + b_ref[...]
    o_ref[...] = jax.nn.sigmoid(normed)


@functools.partial(jax.jit, static_argnames=("block_b",))
def _run(x, h_prev, wt, ut, bias, g, b, block_b):
    B, D_in = x.shape
    H = ut.shape[0]
    grid = (B // block_b,)
    return pl.pallas_call(
        _fused_cell,
        grid=grid,
        in_specs=[
            pl.BlockSpec((block_b, D_in), lambda i: (i, 0)),
            pl.BlockSpec((block_b, H), lambda i: (i, 0)),
            pl.BlockSpec((D_in, H), lambda i: (0, 0)),
            pl.BlockSpec((H, H), lambda i: (0, 0)),
            pl.BlockSpec((1, H), lambda i: (0, 0)),
            pl.BlockSpec((1, H), lambda i: (0, 0)),
            pl.BlockSpec((1, H), lambda i: (0, 0)),
        ],
        out_specs=pl.BlockSpec((block_b, H), lambda i: (i, 0)),
        out_shape=jax.ShapeDtypeStruct((B, H), jnp.float32),
    )(x, h_prev, wt, ut, bias, g, b)


def kernel(x, h_prev, W_w, W_b, U_w, U_b, Q_w, Q_b, M_w, M_b, ln_g, ln_b):
    bias = (W_b + U_b + Q_b).reshape(1, -1)
    return _run(
        x,
        h_prev,
        W_w.T,
        U_w.T,
        bias,
        ln_g.reshape(1, -1),
        ln_b.reshape(1, -1),
        block_b=8192,
    )
